# baseline (speedup 1.0000x reference)
import functools

import jax
import jax.numpy as jnp
from jax import lax
from jax.experimental import pallas as pl
from jax.experimental.pallas import tpu as pltpu
from jax.experimental.pallas import tpu_sc as plsc

_NCH = 2


@functools.lru_cache(maxsize=None)
def _sc_gather(n_total: int, chunk_base: int, n_tokens: int, vocab: int, d: int):
    info = plsc.get_sparse_core_info()
    nw = info.num_cores * info.num_subcores
    per_w = n_tokens // nw
    ch = min(per_w, 32)
    n_ch = per_w // ch
    mesh = plsc.VectorSubcoreMesh(core_axis_name="c", subcore_axis_name="s")

    @functools.partial(
        pl.kernel,
        mesh=mesh,
        out_type=jax.ShapeDtypeStruct((n_tokens, d), jnp.float32),
        scratch_types=[
            pltpu.VMEM((per_w,), jnp.int32),
            pltpu.VMEM((2, ch, d), jnp.float32),
            pltpu.SemaphoreType.DMA,
            pltpu.SemaphoreType.DMA,
        ],
    )
    def gather_kernel(ids_hbm, table_hbm, out_hbm, idx_v, rows_v, gsem, osem):
        wid = lax.axis_index("s") * info.num_cores + lax.axis_index("c")
        base = wid * per_w
        pltpu.sync_copy(ids_hbm.at[pl.ds(chunk_base + base, per_w)], idx_v)

        pltpu.async_copy(table_hbm.at[idx_v.at[pl.ds(0, ch)]], rows_v.at[0], gsem)

        def body(k, _):
            slot = lax.rem(k, 2)
            nxt = lax.rem(k + 1, 2)

            @pl.when(k + 1 < n_ch)
            def _prefetch():
                pltpu.async_copy(
                    table_hbm.at[idx_v.at[pl.ds((k + 1) * ch, ch)]],
                    rows_v.at[nxt],
                    gsem,
                )

            pltpu.make_async_copy(
                table_hbm.at[idx_v.at[pl.ds(k * ch, ch)]], rows_v.at[slot], gsem
            ).wait()
            pltpu.async_copy(
                rows_v.at[slot], out_hbm.at[pl.ds(base + k * ch, ch)], osem
            ).wait()
            return 0

        lax.fori_loop(0, n_ch, body, 0)

    return gather_kernel


def _tc_body(tok_ref, pos_ref, sid_ref, segtab_ref, gamma_ref, beta_ref,
             *rest):
    out_ref = rest[-1]
    x = tok_ref[...] + pos_ref[...]
    s0 = segtab_ref[0:1, :]
    s1 = segtab_ref[1:2, :]
    bb = pl.program_id(0)
    row = sid_ref[pl.ds(bb, 1), :]
    sidc = jnp.transpose(row)
    x = x + s0 + sidc * (s1 - s0)
    mu = jnp.mean(x, axis=1, keepdims=True)
    xc = x - mu
    var = jnp.mean(xc * xc, axis=1, keepdims=True)
    r = lax.rsqrt(var + 1e-6)
    g = gamma_ref[...].reshape(1, -1)
    b = beta_ref[...].reshape(1, -1)
    out_ref[...] = xc * r * g + b


@functools.lru_cache(maxsize=None)
def _tc_chunk(n_tokens: int, seq_len: int, d: int, chunk: int, k: int):
    batch = n_tokens // seq_len
    nch = seq_len // chunk

    in_specs = [
        pl.BlockSpec((chunk, d), lambda bb: (bb, 0)),
        pl.BlockSpec((chunk, d), lambda bb: (k, 0)),
        pl.BlockSpec((batch, chunk), lambda bb: (0, k)),
        pl.BlockSpec((2, d), lambda bb: (0, 0)),
        pl.BlockSpec((d,), lambda bb: (0,)),
        pl.BlockSpec((d,), lambda bb: (0,)),
    ]
    aliases = {}
    if k > 0:
        in_specs.append(pl.BlockSpec((8, 128), lambda bb: (0, 0)))
        aliases = {6: 0}

    return pl.pallas_call(
        _tc_body,
        grid=(batch,),
        in_specs=in_specs,
        out_specs=pl.BlockSpec((chunk, d), lambda bb: (bb * nch + k, 0)),
        out_shape=jax.ShapeDtypeStruct((n_tokens, d), jnp.float32),
        input_output_aliases=aliases,
    )


def kernel(input_ids, segment_ids, token_table, pos_table, seg_table, gamma, beta):
    b, s = input_ids.shape
    vocab, d = token_table.shape
    n = b * s
    cs = s // _NCH
    sid_f = segment_ids.astype(jnp.float32)

    buf = None
    for k in range(_NCH):
        ids_k = lax.slice(input_ids, (0, k * cs), (b, (k + 1) * cs)).reshape(b * cs)
        tok_k = _sc_gather(n, 0, b * cs, vocab, d)(ids_k, token_table)
        args = (tok_k, pos_table, sid_f, seg_table, gamma, beta)
        if k > 0:
            args = args + (buf,)
        buf = _tc_chunk(n, s, d, cs, k)(*args)
    return buf.reshape(b, s, d)


# speedup vs baseline: 1.0314x; 1.0314x over previous
import functools

import jax
import jax.numpy as jnp
from jax import lax
from jax.experimental import pallas as pl
from jax.experimental.pallas import tpu as pltpu
from jax.experimental.pallas import tpu_sc as plsc

_NCH = 1


@functools.lru_cache(maxsize=None)
def _sc_gather(n_total: int, chunk_base: int, n_tokens: int, vocab: int, d: int):
    info = plsc.get_sparse_core_info()
    nw = info.num_cores * info.num_subcores
    per_w = n_tokens // nw
    ch = min(per_w, 32)
    n_ch = per_w // ch
    mesh = plsc.VectorSubcoreMesh(core_axis_name="c", subcore_axis_name="s")

    @functools.partial(
        pl.kernel,
        mesh=mesh,
        out_type=jax.ShapeDtypeStruct((n_tokens, d), jnp.float32),
        scratch_types=[
            pltpu.VMEM((per_w,), jnp.int32),
            pltpu.VMEM((2, ch, d), jnp.float32),
            pltpu.SemaphoreType.DMA,
            pltpu.SemaphoreType.DMA,
        ],
    )
    def gather_kernel(ids_hbm, table_hbm, out_hbm, idx_v, rows_v, gsem, osem):
        wid = lax.axis_index("s") * info.num_cores + lax.axis_index("c")
        base = wid * per_w
        pltpu.sync_copy(ids_hbm.at[pl.ds(chunk_base + base, per_w)], idx_v)

        pltpu.async_copy(table_hbm.at[idx_v.at[pl.ds(0, ch)]], rows_v.at[0], gsem)

        def body(k, _):
            slot = lax.rem(k, 2)
            nxt = lax.rem(k + 1, 2)

            @pl.when(k + 1 < n_ch)
            def _prefetch():
                pltpu.async_copy(
                    table_hbm.at[idx_v.at[pl.ds((k + 1) * ch, ch)]],
                    rows_v.at[nxt],
                    gsem,
                )

            pltpu.make_async_copy(
                table_hbm.at[idx_v.at[pl.ds(k * ch, ch)]], rows_v.at[slot], gsem
            ).wait()
            pltpu.async_copy(
                rows_v.at[slot], out_hbm.at[pl.ds(base + k * ch, ch)], osem
            ).wait()
            return 0

        lax.fori_loop(0, n_ch, body, 0)

    return gather_kernel


def _tc_body(tok_ref, pos_ref, sid_ref, segtab_ref, gamma_ref, beta_ref,
             *rest):
    out_ref = rest[-1]
    x = tok_ref[...] + pos_ref[...]
    s0 = segtab_ref[0:1, :]
    s1 = segtab_ref[1:2, :]
    bb = pl.program_id(0)
    row = sid_ref[pl.ds(bb, 1), :]
    sidc = jnp.transpose(row)
    x = x + s0 + sidc * (s1 - s0)
    mu = jnp.mean(x, axis=1, keepdims=True)
    xc = x - mu
    var = jnp.mean(xc * xc, axis=1, keepdims=True)
    r = lax.rsqrt(var + 1e-6)
    g = gamma_ref[...].reshape(1, -1)
    b = beta_ref[...].reshape(1, -1)
    out_ref[...] = xc * r * g + b


@functools.lru_cache(maxsize=None)
def _tc_chunk(n_tokens: int, seq_len: int, d: int, chunk: int, k: int):
    batch = n_tokens // seq_len
    nch = seq_len // chunk

    in_specs = [
        pl.BlockSpec((chunk, d), lambda bb: (bb, 0)),
        pl.BlockSpec((chunk, d), lambda bb: (k, 0)),
        pl.BlockSpec((batch, chunk), lambda bb: (0, k)),
        pl.BlockSpec((2, d), lambda bb: (0, 0)),
        pl.BlockSpec((d,), lambda bb: (0,)),
        pl.BlockSpec((d,), lambda bb: (0,)),
    ]
    aliases = {}
    if k > 0:
        in_specs.append(pl.BlockSpec((8, 128), lambda bb: (0, 0)))
        aliases = {6: 0}

    return pl.pallas_call(
        _tc_body,
        grid=(batch,),
        in_specs=in_specs,
        out_specs=pl.BlockSpec((chunk, d), lambda bb: (bb * nch + k, 0)),
        out_shape=jax.ShapeDtypeStruct((n_tokens, d), jnp.float32),
        input_output_aliases=aliases,
    )


def kernel(input_ids, segment_ids, token_table, pos_table, seg_table, gamma, beta):
    b, s = input_ids.shape
    vocab, d = token_table.shape
    n = b * s
    cs = s // _NCH
    sid_f = segment_ids.astype(jnp.float32)

    buf = None
    for k in range(_NCH):
        ids_k = lax.slice(input_ids, (0, k * cs), (b, (k + 1) * cs)).reshape(b * cs)
        tok_k = _sc_gather(n, 0, b * cs, vocab, d)(ids_k, token_table)
        args = (tok_k, pos_table, sid_f, seg_table, gamma, beta)
        if k > 0:
            args = args + (buf,)
        buf = _tc_chunk(n, s, d, cs, k)(*args)
    return buf.reshape(b, s, d)


# speedup vs baseline: 1.0404x; 1.0086x over previous
import functools

import jax
import jax.numpy as jnp
from jax import lax
from jax.experimental import pallas as pl
from jax.experimental.pallas import tpu as pltpu
from jax.experimental.pallas import tpu_sc as plsc

_NCH = 1


@functools.lru_cache(maxsize=None)
def _sc_gather(cs: int, k: int, n_tokens: int, vocab: int, d: int):
    info = plsc.get_sparse_core_info()
    nw = info.num_cores * info.num_subcores
    per_w = n_tokens // nw
    ch = min(per_w, 64)
    n_ch = per_w // ch
    mesh = plsc.VectorSubcoreMesh(core_axis_name="c", subcore_axis_name="s")

    @functools.partial(
        pl.kernel,
        mesh=mesh,
        out_type=jax.ShapeDtypeStruct((n_tokens, d), jnp.float32),
        scratch_types=[
            pltpu.VMEM((per_w,), jnp.int32),
            pltpu.VMEM((2, ch, d), jnp.float32),
            pltpu.SemaphoreType.DMA,
            pltpu.SemaphoreType.DMA,
        ],
    )
    def gather_kernel(ids_hbm, table_hbm, out_hbm, idx_v, rows_v, gsem, osem):
        wid = lax.axis_index("s") * info.num_cores + lax.axis_index("c")
        base = wid * per_w
        row = base // cs
        col = k * cs + lax.rem(base, cs)
        pltpu.sync_copy(ids_hbm.at[row, pl.ds(col, per_w)], idx_v)

        pltpu.async_copy(table_hbm.at[idx_v.at[pl.ds(0, ch)]], rows_v.at[0], gsem)

        def body(k, _):
            slot = lax.rem(k, 2)
            nxt = lax.rem(k + 1, 2)

            @pl.when(k > 0)
            def _drain_prev():
                pltpu.make_async_copy(
                    rows_v.at[nxt], out_hbm.at[pl.ds(base + (k - 1) * ch, ch)],
                    osem,
                ).wait()

            @pl.when(k + 1 < n_ch)
            def _prefetch():
                pltpu.async_copy(
                    table_hbm.at[idx_v.at[pl.ds((k + 1) * ch, ch)]],
                    rows_v.at[nxt],
                    gsem,
                )

            pltpu.make_async_copy(
                table_hbm.at[idx_v.at[pl.ds(k * ch, ch)]], rows_v.at[slot], gsem
            ).wait()
            pltpu.async_copy(
                rows_v.at[slot], out_hbm.at[pl.ds(base + k * ch, ch)], osem
            )
            return 0

        lax.fori_loop(0, n_ch, body, 0)
        pltpu.make_async_copy(
            rows_v.at[lax.rem(n_ch - 1, 2)],
            out_hbm.at[pl.ds(base + (n_ch - 1) * ch, ch)],
            osem,
        ).wait()

    return gather_kernel


def _tc_body(tok_ref, pos_ref, sid_ref, segtab_ref, gamma_ref, beta_ref,
             *rest):
    out_ref = rest[-1]
    x = tok_ref[...] + pos_ref[...]
    s0 = segtab_ref[0:1, :]
    s1 = segtab_ref[1:2, :]
    bb = pl.program_id(0)
    row = sid_ref[pl.ds(bb, 1), :]
    sidc = jnp.transpose(row).astype(jnp.float32)
    x = x + s0 + sidc * (s1 - s0)
    mu = jnp.mean(x, axis=1, keepdims=True)
    xc = x - mu
    var = jnp.mean(xc * xc, axis=1, keepdims=True)
    r = lax.rsqrt(var + 1e-6)
    g = gamma_ref[...].reshape(1, -1)
    b = beta_ref[...].reshape(1, -1)
    out_ref[...] = xc * r * g + b


@functools.lru_cache(maxsize=None)
def _tc_chunk(n_tokens: int, seq_len: int, d: int, chunk: int, k: int):
    batch = n_tokens // seq_len
    nch = seq_len // chunk

    in_specs = [
        pl.BlockSpec((chunk, d), lambda bb: (bb, 0)),
        pl.BlockSpec((chunk, d), lambda bb: (k, 0)),
        pl.BlockSpec((batch, chunk), lambda bb: (0, k)),
        pl.BlockSpec((2, d), lambda bb: (0, 0)),
        pl.BlockSpec((d,), lambda bb: (0,)),
        pl.BlockSpec((d,), lambda bb: (0,)),
    ]
    aliases = {}
    if k > 0:
        in_specs.append(pl.BlockSpec((8, 128), lambda bb: (0, 0)))
        aliases = {6: 0}

    return pl.pallas_call(
        _tc_body,
        grid=(batch,),
        in_specs=in_specs,
        out_specs=pl.BlockSpec((chunk, d), lambda bb: (bb * nch + k, 0)),
        out_shape=jax.ShapeDtypeStruct((n_tokens, d), jnp.float32),
        input_output_aliases=aliases,
    )


def kernel(input_ids, segment_ids, token_table, pos_table, seg_table, gamma, beta):
    b, s = input_ids.shape
    vocab, d = token_table.shape
    n = b * s
    cs = s // _NCH

    buf = None
    for k in range(_NCH):
        tok_k = _sc_gather(cs, k, b * cs, vocab, d)(input_ids, token_table)
        args = (tok_k, pos_table, segment_ids, seg_table, gamma, beta)
        if k > 0:
            args = args + (buf,)
        buf = _tc_chunk(n, s, d, cs, k)(*args)
    return buf.reshape(b, s, d)
